# baseline (speedup 1.0000x reference)
import functools

import jax
import jax.numpy as jnp
from jax import lax
from jax.experimental import pallas as pl
from jax.experimental.pallas import tpu as pltpu
from jax.experimental.pallas import tpu_sc as plsc

_NC = 2
_NS = 16
_NW = _NC * _NS

_SC_ROWS = 0
_TC_BLK = 512


def _tc_body(x_ref, pe_ref, o_ref):
    o_ref[...] = x_ref[...] + pe_ref[...][None, :, :]


def _tc_add(x, pe_table):
    B, S, D = x.shape
    return pl.pallas_call(
        _tc_body,
        out_shape=jax.ShapeDtypeStruct((B, S, D), x.dtype),
        grid=(S // _TC_BLK, B),
        in_specs=[
            pl.BlockSpec((1, _TC_BLK, D), lambda s, b: (b, s, 0)),
            pl.BlockSpec((_TC_BLK, D), lambda s, b: (s, 0)),
        ],
        out_specs=pl.BlockSpec((1, _TC_BLK, D), lambda s, b: (b, s, 0)),
    )(x, pe_table)


def _sc_add(xf, pef, n_rows, D, x_row0, pe_row0):
    rpw = n_rows // _NW
    TILE = 16
    n_tiles = rpw // TILE
    UNROLL = 8

    @functools.partial(
        pl.kernel,
        out_type=jax.ShapeDtypeStruct((n_rows * D,), jnp.float32),
        mesh=plsc.VectorSubcoreMesh(core_axis_name="c", subcore_axis_name="s"),
        scratch_types=[
            pltpu.VMEM((2, TILE * D), jnp.float32),
            pltpu.VMEM((2, TILE * D), jnp.float32),
            pltpu.VMEM((2, TILE * D), jnp.float32),
            pltpu.SemaphoreType.DMA,
            pltpu.SemaphoreType.DMA,
        ],
    )
    def k(x_hbm, pe_hbm, out_hbm, xbuf, pebuf, obuf, insem, outsem):
        wid = lax.axis_index("s") * _NC + lax.axis_index("c")
        base = wid * (rpw * D)

        def in_copies(t, slot):
            off = base + t * (TILE * D)
            cx = pltpu.make_async_copy(
                x_hbm.at[pl.ds(x_row0 * D + off, TILE * D)], xbuf.at[slot],
                insem)
            cp = pltpu.make_async_copy(
                pe_hbm.at[pl.ds(pe_row0 * D + off, TILE * D)], pebuf.at[slot],
                insem)
            return cx, cp

        def out_copy(t, slot):
            off = base + t * (TILE * D)
            return pltpu.make_async_copy(
                obuf.at[slot], out_hbm.at[pl.ds(off, TILE * D)], outsem)

        for s in (0, 1):
            cx, cp = in_copies(s, s)
            cx.start()
            cp.start()

        def tile_body(t, slot):
            @pl.when(t >= 2)
            def _drain():
                out_copy(t - 2, slot).wait()

            cx, cp = in_copies(t, slot)
            cx.wait()
            cp.wait()

            def addv(i, carry):
                o = i * (16 * UNROLL)
                for u in range(UNROLL):
                    q = o + u * 16
                    obuf[slot, pl.ds(q, 16)] = (
                        xbuf[slot, pl.ds(q, 16)] + pebuf[slot, pl.ds(q, 16)])
                return carry

            lax.fori_loop(0, (TILE * D) // (16 * UNROLL), addv, 0)
            out_copy(t, slot).start()

            @pl.when(t + 2 < n_tiles)
            def _refill():
                nx, np_ = in_copies(t + 2, slot)
                nx.start()
                np_.start()

        def pair_body(p, carry):
            tile_body(2 * p, 0)
            tile_body(2 * p + 1, 1)
            return carry

        lax.fori_loop(0, n_tiles // 2, pair_body, 0)
        out_copy(n_tiles - 2, 0).wait()
        out_copy(n_tiles - 1, 1).wait()

    return k(xf, pef)


def kernel(x, pe_table):
    return _tc_add(x, pe_table)


# speedup vs baseline: 1.1840x; 1.1840x over previous
import functools

import jax
import jax.numpy as jnp
from jax import lax
from jax.experimental import pallas as pl
from jax.experimental.pallas import tpu as pltpu
from jax.experimental.pallas import tpu_sc as plsc

_NC = 2
_NS = 16
_NW = _NC * _NS

_SC_ROWS = 0
_TC_BLK = 512


def _tc_body(x_ref, pe_ref, o_ref):
    i = pl.program_id(0)
    o_ref[...] = x_ref[...] + pe_ref[pl.ds(i * _TC_BLK, _TC_BLK), :][None, :, :]


def _tc_add(x, pe_table):
    B, S, D = x.shape
    return pl.pallas_call(
        _tc_body,
        out_shape=jax.ShapeDtypeStruct((B, S, D), x.dtype),
        grid=(S // _TC_BLK,),
        in_specs=[
            pl.BlockSpec((B, _TC_BLK, D), lambda i: (0, i, 0)),
            pl.BlockSpec((S, D), lambda i: (0, 0)),
        ],
        out_specs=pl.BlockSpec((B, _TC_BLK, D), lambda i: (0, i, 0)),
    )(x, pe_table)


def _sc_add(xf, pef, n_rows, D, x_row0, pe_row0):
    rpw = n_rows // _NW
    TILE = 16
    n_tiles = rpw // TILE
    UNROLL = 8

    @functools.partial(
        pl.kernel,
        out_type=jax.ShapeDtypeStruct((n_rows * D,), jnp.float32),
        mesh=plsc.VectorSubcoreMesh(core_axis_name="c", subcore_axis_name="s"),
        scratch_types=[
            pltpu.VMEM((2, TILE * D), jnp.float32),
            pltpu.VMEM((2, TILE * D), jnp.float32),
            pltpu.VMEM((2, TILE * D), jnp.float32),
            pltpu.SemaphoreType.DMA,
            pltpu.SemaphoreType.DMA,
        ],
    )
    def k(x_hbm, pe_hbm, out_hbm, xbuf, pebuf, obuf, insem, outsem):
        wid = lax.axis_index("s") * _NC + lax.axis_index("c")
        base = wid * (rpw * D)

        def in_copies(t, slot):
            off = base + t * (TILE * D)
            cx = pltpu.make_async_copy(
                x_hbm.at[pl.ds(x_row0 * D + off, TILE * D)], xbuf.at[slot],
                insem)
            cp = pltpu.make_async_copy(
                pe_hbm.at[pl.ds(pe_row0 * D + off, TILE * D)], pebuf.at[slot],
                insem)
            return cx, cp

        def out_copy(t, slot):
            off = base + t * (TILE * D)
            return pltpu.make_async_copy(
                obuf.at[slot], out_hbm.at[pl.ds(off, TILE * D)], outsem)

        for s in (0, 1):
            cx, cp = in_copies(s, s)
            cx.start()
            cp.start()

        def tile_body(t, slot):
            @pl.when(t >= 2)
            def _drain():
                out_copy(t - 2, slot).wait()

            cx, cp = in_copies(t, slot)
            cx.wait()
            cp.wait()

            def addv(i, carry):
                o = i * (16 * UNROLL)
                for u in range(UNROLL):
                    q = o + u * 16
                    obuf[slot, pl.ds(q, 16)] = (
                        xbuf[slot, pl.ds(q, 16)] + pebuf[slot, pl.ds(q, 16)])
                return carry

            lax.fori_loop(0, (TILE * D) // (16 * UNROLL), addv, 0)
            out_copy(t, slot).start()

            @pl.when(t + 2 < n_tiles)
            def _refill():
                nx, np_ = in_copies(t + 2, slot)
                nx.start()
                np_.start()

        def pair_body(p, carry):
            tile_body(2 * p, 0)
            tile_body(2 * p + 1, 1)
            return carry

        lax.fori_loop(0, n_tiles // 2, pair_body, 0)
        out_copy(n_tiles - 2, 0).wait()
        out_copy(n_tiles - 1, 1).wait()

    return k(xf, pef)


def kernel(x, pe_table):
    return _tc_add(x, pe_table)


# speedup vs baseline: 1.1916x; 1.0063x over previous
import functools

import jax
import jax.numpy as jnp
from jax import lax
from jax.experimental import pallas as pl
from jax.experimental.pallas import tpu as pltpu
from jax.experimental.pallas import tpu_sc as plsc

_NC = 2
_NS = 16
_NW = _NC * _NS

_SC_ROWS = 0
_TC_BLK = 768


def _tc_body(x_ref, pe_ref, o_ref):
    o_ref[...] = x_ref[...] + pe_ref[...][None, :, :]


def _tc_add(x, pe_table):
    B, S, D = x.shape
    n = pl.cdiv(S, _TC_BLK)
    return pl.pallas_call(
        _tc_body,
        out_shape=jax.ShapeDtypeStruct((B, S, D), x.dtype),
        grid=(n,),
        in_specs=[
            pl.BlockSpec((B, _TC_BLK, D), lambda i: (0, i, 0)),
            pl.BlockSpec((_TC_BLK, D), lambda i: (i, 0)),
        ],
        out_specs=pl.BlockSpec((B, _TC_BLK, D), lambda i: (0, i, 0)),
    )(x, pe_table)


def _sc_add(xf, pef, n_rows, D, x_row0, pe_row0):
    rpw = n_rows // _NW
    TILE = 16
    n_tiles = rpw // TILE
    UNROLL = 8

    @functools.partial(
        pl.kernel,
        out_type=jax.ShapeDtypeStruct((n_rows * D,), jnp.float32),
        mesh=plsc.VectorSubcoreMesh(core_axis_name="c", subcore_axis_name="s"),
        scratch_types=[
            pltpu.VMEM((2, TILE * D), jnp.float32),
            pltpu.VMEM((2, TILE * D), jnp.float32),
            pltpu.VMEM((2, TILE * D), jnp.float32),
            pltpu.SemaphoreType.DMA,
            pltpu.SemaphoreType.DMA,
        ],
    )
    def k(x_hbm, pe_hbm, out_hbm, xbuf, pebuf, obuf, insem, outsem):
        wid = lax.axis_index("s") * _NC + lax.axis_index("c")
        base = wid * (rpw * D)

        def in_copies(t, slot):
            off = base + t * (TILE * D)
            cx = pltpu.make_async_copy(
                x_hbm.at[pl.ds(x_row0 * D + off, TILE * D)], xbuf.at[slot],
                insem)
            cp = pltpu.make_async_copy(
                pe_hbm.at[pl.ds(pe_row0 * D + off, TILE * D)], pebuf.at[slot],
                insem)
            return cx, cp

        def out_copy(t, slot):
            off = base + t * (TILE * D)
            return pltpu.make_async_copy(
                obuf.at[slot], out_hbm.at[pl.ds(off, TILE * D)], outsem)

        for s in (0, 1):
            cx, cp = in_copies(s, s)
            cx.start()
            cp.start()

        def tile_body(t, slot):
            @pl.when(t >= 2)
            def _drain():
                out_copy(t - 2, slot).wait()

            cx, cp = in_copies(t, slot)
            cx.wait()
            cp.wait()

            def addv(i, carry):
                o = i * (16 * UNROLL)
                for u in range(UNROLL):
                    q = o + u * 16
                    obuf[slot, pl.ds(q, 16)] = (
                        xbuf[slot, pl.ds(q, 16)] + pebuf[slot, pl.ds(q, 16)])
                return carry

            lax.fori_loop(0, (TILE * D) // (16 * UNROLL), addv, 0)
            out_copy(t, slot).start()

            @pl.when(t + 2 < n_tiles)
            def _refill():
                nx, np_ = in_copies(t + 2, slot)
                nx.start()
                np_.start()

        def pair_body(p, carry):
            tile_body(2 * p, 0)
            tile_body(2 * p + 1, 1)
            return carry

        lax.fori_loop(0, n_tiles // 2, pair_body, 0)
        out_copy(n_tiles - 2, 0).wait()
        out_copy(n_tiles - 1, 1).wait()

    return k(xf, pef)


def kernel(x, pe_table):
    return _tc_add(x, pe_table)


# speedup vs baseline: 1.2027x; 1.0093x over previous
import jax
import jax.numpy as jnp
from jax.experimental import pallas as pl

_BLK = 768


def _add_body(x_ref, pe_ref, o_ref):
    o_ref[...] = x_ref[...] + pe_ref[...][None, :, :]


def kernel(x, pe_table):
    B, S, D = x.shape
    return pl.pallas_call(
        _add_body,
        out_shape=jax.ShapeDtypeStruct((B, S, D), x.dtype),
        grid=(pl.cdiv(S, _BLK),),
        in_specs=[
            pl.BlockSpec((B, _BLK, D), lambda i: (0, i, 0)),
            pl.BlockSpec((_BLK, D), lambda i: (i, 0)),
        ],
        out_specs=pl.BlockSpec((B, _BLK, D), lambda i: (0, i, 0)),
    )(x, pe_table)


# speedup vs baseline: 1.2497x; 1.0391x over previous
import jax
import jax.numpy as jnp
from jax.experimental import pallas as pl
from jax.experimental.pallas import tpu as pltpu

_BLK = 896


def _add_body(x_ref, pe_ref, o_ref):
    o_ref[...] = x_ref[...] + pe_ref[...][None, :, :]


def kernel(x, pe_table):
    B, S, D = x.shape
    return pl.pallas_call(
        _add_body,
        out_shape=jax.ShapeDtypeStruct((B, S, D), x.dtype),
        grid=(pl.cdiv(S, _BLK),),
        in_specs=[
            pl.BlockSpec((B, _BLK, D), lambda i: (0, i, 0)),
            pl.BlockSpec((_BLK, D), lambda i: (i, 0)),
        ],
        out_specs=pl.BlockSpec((B, _BLK, D), lambda i: (0, i, 0)),
        compiler_params=pltpu.CompilerParams(vmem_limit_bytes=67108864),
    )(x, pe_table)
